# Initial kernel scaffold; baseline (speedup 1.0000x reference)
import jax
import jax.numpy as jnp
from jax.experimental import pallas as pl


def kernel(xyz1, xyz2, points1, points2, W0, b0, g0, beta0, W1, b1, g1, beta1):
    raise NotImplementedError("write your pallas kernel here")


# speedup vs baseline: 14.5731x; 14.5731x over previous
import functools

import jax
import jax.numpy as jnp
from jax import lax
from jax.experimental import pallas as pl
from jax.experimental.pallas import tpu as pltpu
from jax.experimental.pallas import tpu_sc as plsc

B, N, S = 8, 4096, 1024
D1, D2 = 128, 256
C1, C2 = 256, 256
BN = B * N

TN = 512
NT = N // TN


def _knn_body(x1_ref, x2_ref, idx_ref, w_ref):
    b = pl.program_id(0)
    x1 = x1_ref[0]
    x2 = x2_ref[0]
    d = lax.dot_general(x1, x2, (((0,), (0,)), ((), ())),
                        preferred_element_type=jnp.float32)
    d = -2.0 * d
    d = d + jnp.sum(x1 * x1, axis=0)[:, None]
    d = d + jnp.sum(x2 * x2, axis=0)[None, :]
    iota = lax.broadcasted_iota(jnp.int32, (TN, S), 1)
    recips = []
    idxs = []
    for k in range(3):
        m = jnp.min(d, axis=1, keepdims=True)
        ik = jnp.min(jnp.where(d == m, iota, S), axis=1, keepdims=True)
        recips.append(1.0 / (m + 1e-8))
        idxs.append(ik)
        d = jnp.where(iota == ik, jnp.float32(jnp.inf), d)
    norm = recips[0] + recips[1] + recips[2]
    for k in range(3):
        idx_ref[0, :, k:k + 1] = idxs[k] + b * S
        w_ref[0, :, k:k + 1] = recips[k] / norm


def _knn(xyz1, xyz2):
    return pl.pallas_call(
        _knn_body,
        grid=(B, NT),
        in_specs=[
            pl.BlockSpec((1, 3, TN), lambda b, t: (b, 0, t)),
            pl.BlockSpec((1, 3, S), lambda b, t: (b, 0, 0)),
        ],
        out_specs=[
            pl.BlockSpec((1, TN, 3), lambda b, t: (b, t, 0)),
            pl.BlockSpec((1, TN, 3), lambda b, t: (b, t, 0)),
        ],
        out_shape=[
            jax.ShapeDtypeStruct((B, N, 3), jnp.int32),
            jax.ShapeDtypeStruct((B, N, 3), jnp.float32),
        ],
    )(xyz1, xyz2)


_NW = 32
_PW = BN // _NW
_CP = 16
_NCH = _PW // _CP


def _interp_sc(table, idx_flat, w_flat):
    mesh = plsc.VectorSubcoreMesh(core_axis_name="c", subcore_axis_name="s")

    @functools.partial(
        pl.kernel,
        mesh=mesh,
        compiler_params=pltpu.CompilerParams(needs_layout_passes=False),
        out_type=jax.ShapeDtypeStruct((BN, D2), jnp.float32),
        scratch_types=[
            pltpu.VMEM((_PW * 3,), jnp.int32),
            pltpu.VMEM((_PW * 3,), jnp.float32),
            pltpu.VMEM((_CP * 3, D2), jnp.float32),
            pltpu.VMEM((_CP, D2), jnp.float32),
            pltpu.SemaphoreType.DMA,
        ],
    )
    def k(table_hbm, idx_hbm, w_hbm, out_hbm, idx_v, w_v, rows_v, outb_v, sem):
        wid = lax.axis_index("s") * 2 + lax.axis_index("c")
        base = wid * _PW
        pltpu.sync_copy(idx_hbm.at[pl.ds(base * 3, _PW * 3)], idx_v)
        pltpu.sync_copy(w_hbm.at[pl.ds(base * 3, _PW * 3)], w_v)

        def chunk(ci, carry):
            cb = ci * _CP
            pltpu.async_copy(
                table_hbm.at[idx_v.at[pl.ds(cb * 3, _CP * 3)]], rows_v, sem
            ).wait()
            for p in range(_CP):
                wb = (cb + p) * 3
                w0 = plsc.load_gather(w_v, [jnp.full((16,), 0, jnp.int32) + wb])
                w1 = plsc.load_gather(w_v, [jnp.full((16,), 1, jnp.int32) + wb])
                w2 = plsc.load_gather(w_v, [jnp.full((16,), 2, jnp.int32) + wb])
                for j in range(D2 // 16):
                    sl = pl.ds(j * 16, 16)
                    outb_v[p, sl] = (rows_v[3 * p, sl] * w0
                                     + rows_v[3 * p + 1, sl] * w1
                                     + rows_v[3 * p + 2, sl] * w2)
            pltpu.sync_copy(outb_v, out_hbm.at[pl.ds(base + cb, _CP)])
            return carry

        lax.fori_loop(0, _NCH, chunk, 0)

    return k(table, idx_flat, w_flat)


def _m1_body(p1_ref, it_ref, wa_ref, wb_ref, b0_ref, z_ref, s_ref, q_ref):
    b = pl.program_id(0)
    t = pl.program_id(1)
    z = (jnp.dot(wa_ref[...], p1_ref[0], preferred_element_type=jnp.float32)
         + lax.dot_general(wb_ref[...], it_ref[...], (((1,), (1,)), ((), ())),
                           preferred_element_type=jnp.float32)
         + b0_ref[...])
    z_ref[0] = z

    @pl.when((b == 0) & (t == 0))
    def _():
        s_ref[...] = jnp.zeros_like(s_ref)
        q_ref[...] = jnp.zeros_like(q_ref)

    s_ref[...] += jnp.sum(z, axis=1, keepdims=True)
    q_ref[...] += jnp.sum(z * z, axis=1, keepdims=True)


def _m1(points1, interp, wa, wb, b0c):
    return pl.pallas_call(
        _m1_body,
        grid=(B, NT),
        in_specs=[
            pl.BlockSpec((1, D1, TN), lambda b, t: (b, 0, t)),
            pl.BlockSpec((TN, D2), lambda b, t: (b * NT + t, 0)),
            pl.BlockSpec((C1, D1), lambda b, t: (0, 0)),
            pl.BlockSpec((C1, D2), lambda b, t: (0, 0)),
            pl.BlockSpec((C1, 1), lambda b, t: (0, 0)),
        ],
        out_specs=[
            pl.BlockSpec((1, C1, TN), lambda b, t: (b, 0, t)),
            pl.BlockSpec((C1, 1), lambda b, t: (0, 0)),
            pl.BlockSpec((C1, 1), lambda b, t: (0, 0)),
        ],
        out_shape=[
            jax.ShapeDtypeStruct((B, C1, N), jnp.float32),
            jax.ShapeDtypeStruct((C1, 1), jnp.float32),
            jax.ShapeDtypeStruct((C1, 1), jnp.float32),
        ],
    )(points1, interp, wa, wb, b0c)


def _m2_body(z_ref, s_ref, q_ref, g_ref, be_ref, w1_ref, b1_ref,
             z2_ref, s2_ref, q2_ref):
    b = pl.program_id(0)
    t = pl.program_id(1)
    mean = s_ref[...] * (1.0 / BN)
    var = q_ref[...] * (1.0 / BN) - mean * mean
    inv = lax.rsqrt(var + 1e-5)
    h = (z_ref[0] - mean) * (inv * g_ref[...]) + be_ref[...]
    h = jnp.maximum(h, 0.0)
    z2 = jnp.dot(w1_ref[...], h, preferred_element_type=jnp.float32) + b1_ref[...]
    z2_ref[0] = z2

    @pl.when((b == 0) & (t == 0))
    def _():
        s2_ref[...] = jnp.zeros_like(s2_ref)
        q2_ref[...] = jnp.zeros_like(q2_ref)

    s2_ref[...] += jnp.sum(z2, axis=1, keepdims=True)
    q2_ref[...] += jnp.sum(z2 * z2, axis=1, keepdims=True)


def _m2(z1, s1, q1, g0c, be0c, w1, b1c):
    return pl.pallas_call(
        _m2_body,
        grid=(B, NT),
        in_specs=[
            pl.BlockSpec((1, C1, TN), lambda b, t: (b, 0, t)),
            pl.BlockSpec((C1, 1), lambda b, t: (0, 0)),
            pl.BlockSpec((C1, 1), lambda b, t: (0, 0)),
            pl.BlockSpec((C1, 1), lambda b, t: (0, 0)),
            pl.BlockSpec((C1, 1), lambda b, t: (0, 0)),
            pl.BlockSpec((C2, C1), lambda b, t: (0, 0)),
            pl.BlockSpec((C2, 1), lambda b, t: (0, 0)),
        ],
        out_specs=[
            pl.BlockSpec((1, C2, TN), lambda b, t: (b, 0, t)),
            pl.BlockSpec((C2, 1), lambda b, t: (0, 0)),
            pl.BlockSpec((C2, 1), lambda b, t: (0, 0)),
        ],
        out_shape=[
            jax.ShapeDtypeStruct((B, C2, N), jnp.float32),
            jax.ShapeDtypeStruct((C2, 1), jnp.float32),
            jax.ShapeDtypeStruct((C2, 1), jnp.float32),
        ],
    )(z1, s1, q1, g0c, be0c, w1, b1c)


def _m3_body(z_ref, s_ref, q_ref, g_ref, be_ref, o_ref):
    mean = s_ref[...] * (1.0 / BN)
    var = q_ref[...] * (1.0 / BN) - mean * mean
    inv = lax.rsqrt(var + 1e-5)
    o_ref[0] = jnp.maximum(
        (z_ref[0] - mean) * (inv * g_ref[...]) + be_ref[...], 0.0)


def _m3(z2, s2, q2, g1c, be1c):
    return pl.pallas_call(
        _m3_body,
        grid=(B, NT),
        in_specs=[
            pl.BlockSpec((1, C2, TN), lambda b, t: (b, 0, t)),
            pl.BlockSpec((C2, 1), lambda b, t: (0, 0)),
            pl.BlockSpec((C2, 1), lambda b, t: (0, 0)),
            pl.BlockSpec((C2, 1), lambda b, t: (0, 0)),
            pl.BlockSpec((C2, 1), lambda b, t: (0, 0)),
        ],
        out_specs=pl.BlockSpec((1, C2, TN), lambda b, t: (b, 0, t)),
        out_shape=jax.ShapeDtypeStruct((B, C2, N), jnp.float32),
    )(z2, s2, q2, g1c, be1c)


def kernel(xyz1, xyz2, points1, points2, W0, b0, g0, beta0, W1, b1, g1, beta1):
    idx3, w3 = _knn(xyz1, xyz2)
    table = jnp.transpose(points2, (0, 2, 1)).reshape(B * S, D2)
    interp = _interp_sc(table, idx3.reshape(-1), w3.reshape(-1))
    z1, s1, q1 = _m1(points1, interp, W0[:, :D1], W0[:, D1:],
                     b0.reshape(-1, 1))
    z2, s2, q2 = _m2(z1, s1, q1, g0.reshape(-1, 1), beta0.reshape(-1, 1),
                     W1, b1.reshape(-1, 1))
    return _m3(z2, s2, q2, g1.reshape(-1, 1), beta1.reshape(-1, 1))


# speedup vs baseline: 15.4422x; 1.0596x over previous
import functools

import jax
import jax.numpy as jnp
from jax import lax
from jax.experimental import pallas as pl
from jax.experimental.pallas import tpu as pltpu
from jax.experimental.pallas import tpu_sc as plsc

B, N, S = 8, 4096, 1024
D1, D2 = 128, 256
C1, C2 = 256, 256
BN = B * N

TN = 512
NT = N // TN


def _knn_body(x1_ref, x2_ref, idx_ref, w_ref):
    b = pl.program_id(0)
    x1 = x1_ref[0]
    x2 = x2_ref[0]
    d = lax.dot_general(x1, x2, (((0,), (0,)), ((), ())),
                        preferred_element_type=jnp.float32)
    d = -2.0 * d
    d = d + jnp.sum(x1 * x1, axis=0)[:, None]
    d = d + jnp.sum(x2 * x2, axis=0)[None, :]
    iota = lax.broadcasted_iota(jnp.int32, (TN, S), 1)
    recips = []
    idxs = []
    for k in range(3):
        m = jnp.min(d, axis=1, keepdims=True)
        ik = jnp.min(jnp.where(d == m, iota, S), axis=1, keepdims=True)
        recips.append(1.0 / (m + 1e-8))
        idxs.append(ik)
        if k < 2:
            d = jnp.where(iota == ik, jnp.float32(jnp.inf), d)
    norm = recips[0] + recips[1] + recips[2]
    for k in range(3):
        idx_ref[:, k:k + 1] = idxs[k] + b * S
        w_ref[:, k:k + 1] = recips[k] / norm


def _knn(xyz1, xyz2):
    return pl.pallas_call(
        _knn_body,
        grid=(B, NT),
        in_specs=[
            pl.BlockSpec((1, 3, TN), lambda b, t: (b, 0, t)),
            pl.BlockSpec((1, 3, S), lambda b, t: (b, 0, 0)),
        ],
        out_specs=[
            pl.BlockSpec((TN, 3), lambda b, t: (b * NT + t, 0)),
            pl.BlockSpec((TN, 3), lambda b, t: (b * NT + t, 0)),
        ],
        out_shape=[
            jax.ShapeDtypeStruct((BN, 3), jnp.int32),
            jax.ShapeDtypeStruct((BN, 3), jnp.float32),
        ],
    )(xyz1, xyz2)


_NW = 32
_PW = BN // _NW
_CP = 16
_NCH = _PW // _CP


def _interp_sc(table, idx_flat, w_flat):
    mesh = plsc.VectorSubcoreMesh(core_axis_name="c", subcore_axis_name="s")

    @functools.partial(
        pl.kernel,
        mesh=mesh,
        compiler_params=pltpu.CompilerParams(needs_layout_passes=False),
        out_type=jax.ShapeDtypeStruct((BN, D2), jnp.float32),
        scratch_types=[
            pltpu.VMEM((_PW * 3,), jnp.int32),
            pltpu.VMEM((_PW * 3,), jnp.float32),
            pltpu.VMEM((_CP * 3, D2), jnp.float32),
            pltpu.VMEM((_CP * 3, D2), jnp.float32),
            pltpu.VMEM((_CP, D2), jnp.float32),
            pltpu.VMEM((_CP, D2), jnp.float32),
            pltpu.SemaphoreType.DMA,
            pltpu.SemaphoreType.DMA,
        ],
    )
    def k(table_hbm, idx_hbm, w_hbm, out_hbm, idx_v, w_v,
          rows0, rows1, outb0, outb1, sem0, sem1):
        wid = lax.axis_index("s") * 2 + lax.axis_index("c")
        base = wid * _PW
        pltpu.sync_copy(idx_hbm.at[pl.ds(base * 3, _PW * 3)], idx_v)
        pltpu.sync_copy(w_hbm.at[pl.ds(base * 3, _PW * 3)], w_v)

        def gather(ci, rows, sem):
            return pltpu.async_copy(
                table_hbm.at[idx_v.at[pl.ds(ci * _CP * 3, _CP * 3)]], rows, sem)

        def compute(ci, rows, outb):
            cb = ci * _CP
            for p in range(_CP):
                wb = (cb + p) * 3
                w0 = plsc.load_gather(w_v, [jnp.full((16,), 0, jnp.int32) + wb])
                w1 = plsc.load_gather(w_v, [jnp.full((16,), 1, jnp.int32) + wb])
                w2 = plsc.load_gather(w_v, [jnp.full((16,), 2, jnp.int32) + wb])
                for j in range(D2 // 16):
                    sl = pl.ds(j * 16, 16)
                    outb[p, sl] = (rows[3 * p, sl] * w0
                                   + rows[3 * p + 1, sl] * w1
                                   + rows[3 * p + 2, sl] * w2)
            pltpu.sync_copy(outb, out_hbm.at[pl.ds(base + cb, _CP)])

        def drain(rows, sem):
            pltpu.make_async_copy(
                table_hbm.at[pl.ds(0, _CP * 3)], rows, sem).wait()

        gather(0, rows0, sem0)
        gather(1, rows1, sem1)

        def pair(i, carry):
            c = 2 * i
            drain(rows0, sem0)
            compute(c, rows0, outb0)
            gather(c + 2, rows0, sem0)
            drain(rows1, sem1)
            compute(c + 1, rows1, outb1)
            gather(c + 3, rows1, sem1)
            return carry

        lax.fori_loop(0, _NCH // 2 - 1, pair, 0)
        drain(rows0, sem0)
        compute(_NCH - 2, rows0, outb0)
        drain(rows1, sem1)
        compute(_NCH - 1, rows1, outb1)

    return k(table, idx_flat, w_flat)


def _m1_body(p1_ref, it_ref, wa_ref, wb_ref, b0_ref, z_ref, s_ref, q_ref):
    b = pl.program_id(0)
    t = pl.program_id(1)
    z = (jnp.dot(wa_ref[...], p1_ref[0], preferred_element_type=jnp.float32)
         + lax.dot_general(wb_ref[...], it_ref[...], (((1,), (1,)), ((), ())),
                           preferred_element_type=jnp.float32)
         + b0_ref[...])
    z_ref[0] = z

    @pl.when((b == 0) & (t == 0))
    def _():
        s_ref[...] = jnp.zeros_like(s_ref)
        q_ref[...] = jnp.zeros_like(q_ref)

    s_ref[...] += jnp.sum(z, axis=1, keepdims=True)
    q_ref[...] += jnp.sum(z * z, axis=1, keepdims=True)


def _m1(points1, interp, wa, wb, b0c):
    return pl.pallas_call(
        _m1_body,
        grid=(B, NT),
        in_specs=[
            pl.BlockSpec((1, D1, TN), lambda b, t: (b, 0, t)),
            pl.BlockSpec((TN, D2), lambda b, t: (b * NT + t, 0)),
            pl.BlockSpec((C1, D1), lambda b, t: (0, 0)),
            pl.BlockSpec((C1, D2), lambda b, t: (0, 0)),
            pl.BlockSpec((C1, 1), lambda b, t: (0, 0)),
        ],
        out_specs=[
            pl.BlockSpec((1, C1, TN), lambda b, t: (b, 0, t)),
            pl.BlockSpec((C1, 1), lambda b, t: (0, 0)),
            pl.BlockSpec((C1, 1), lambda b, t: (0, 0)),
        ],
        out_shape=[
            jax.ShapeDtypeStruct((B, C1, N), jnp.float32),
            jax.ShapeDtypeStruct((C1, 1), jnp.float32),
            jax.ShapeDtypeStruct((C1, 1), jnp.float32),
        ],
    )(points1, interp, wa, wb, b0c)


def _m2_body(z_ref, s_ref, q_ref, g_ref, be_ref, w1_ref, b1_ref,
             z2_ref, s2_ref, q2_ref):
    b = pl.program_id(0)
    t = pl.program_id(1)
    mean = s_ref[...] * (1.0 / BN)
    var = q_ref[...] * (1.0 / BN) - mean * mean
    inv = lax.rsqrt(var + 1e-5)
    h = (z_ref[0] - mean) * (inv * g_ref[...]) + be_ref[...]
    h = jnp.maximum(h, 0.0)
    z2 = jnp.dot(w1_ref[...], h, preferred_element_type=jnp.float32) + b1_ref[...]
    z2_ref[0] = z2

    @pl.when((b == 0) & (t == 0))
    def _():
        s2_ref[...] = jnp.zeros_like(s2_ref)
        q2_ref[...] = jnp.zeros_like(q2_ref)

    s2_ref[...] += jnp.sum(z2, axis=1, keepdims=True)
    q2_ref[...] += jnp.sum(z2 * z2, axis=1, keepdims=True)


def _m2(z1, s1, q1, g0c, be0c, w1, b1c):
    return pl.pallas_call(
        _m2_body,
        grid=(B, NT),
        in_specs=[
            pl.BlockSpec((1, C1, TN), lambda b, t: (b, 0, t)),
            pl.BlockSpec((C1, 1), lambda b, t: (0, 0)),
            pl.BlockSpec((C1, 1), lambda b, t: (0, 0)),
            pl.BlockSpec((C1, 1), lambda b, t: (0, 0)),
            pl.BlockSpec((C1, 1), lambda b, t: (0, 0)),
            pl.BlockSpec((C2, C1), lambda b, t: (0, 0)),
            pl.BlockSpec((C2, 1), lambda b, t: (0, 0)),
        ],
        out_specs=[
            pl.BlockSpec((1, C2, TN), lambda b, t: (b, 0, t)),
            pl.BlockSpec((C2, 1), lambda b, t: (0, 0)),
            pl.BlockSpec((C2, 1), lambda b, t: (0, 0)),
        ],
        out_shape=[
            jax.ShapeDtypeStruct((B, C2, N), jnp.float32),
            jax.ShapeDtypeStruct((C2, 1), jnp.float32),
            jax.ShapeDtypeStruct((C2, 1), jnp.float32),
        ],
    )(z1, s1, q1, g0c, be0c, w1, b1c)


def _m3_body(z_ref, s_ref, q_ref, g_ref, be_ref, o_ref):
    mean = s_ref[...] * (1.0 / BN)
    var = q_ref[...] * (1.0 / BN) - mean * mean
    inv = lax.rsqrt(var + 1e-5)
    o_ref[0] = jnp.maximum(
        (z_ref[0] - mean) * (inv * g_ref[...]) + be_ref[...], 0.0)


def _m3(z2, s2, q2, g1c, be1c):
    return pl.pallas_call(
        _m3_body,
        grid=(B, NT),
        in_specs=[
            pl.BlockSpec((1, C2, TN), lambda b, t: (b, 0, t)),
            pl.BlockSpec((C2, 1), lambda b, t: (0, 0)),
            pl.BlockSpec((C2, 1), lambda b, t: (0, 0)),
            pl.BlockSpec((C2, 1), lambda b, t: (0, 0)),
            pl.BlockSpec((C2, 1), lambda b, t: (0, 0)),
        ],
        out_specs=pl.BlockSpec((1, C2, TN), lambda b, t: (b, 0, t)),
        out_shape=jax.ShapeDtypeStruct((B, C2, N), jnp.float32),
    )(z2, s2, q2, g1c, be1c)


def kernel(xyz1, xyz2, points1, points2, W0, b0, g0, beta0, W1, b1, g1, beta1):
    idx3, w3 = _knn(xyz1, xyz2)
    table = jnp.transpose(points2, (0, 2, 1)).reshape(B * S, D2)
    interp = _interp_sc(table, idx3.reshape(-1), w3.reshape(-1))
    z1, s1, q1 = _m1(points1, interp, W0[:, :D1], W0[:, D1:],
                     b0.reshape(-1, 1))
    z2, s2, q2 = _m2(z1, s1, q1, g0.reshape(-1, 1), beta0.reshape(-1, 1),
                     W1, b1.reshape(-1, 1))
    return _m3(z2, s2, q2, g1.reshape(-1, 1), beta1.reshape(-1, 1))


# speedup vs baseline: 18.2278x; 1.1804x over previous
import functools

import jax
import jax.numpy as jnp
from jax import lax
from jax.experimental import pallas as pl
from jax.experimental.pallas import tpu as pltpu
from jax.experimental.pallas import tpu_sc as plsc

B, N, S = 8, 4096, 1024
D1, D2 = 128, 256
C1, C2 = 256, 256
BN = B * N

TN = 512
NT = N // TN
TK = 1024
NK = N // TK


def _knn_body(x1_ref, x2_ref, idx_ref, w_ref):
    b = pl.program_id(0)
    x1 = x1_ref[0]
    x2 = x2_ref[0]
    d = lax.dot_general(x2, x1, (((0,), (0,)), ((), ())),
                        preferred_element_type=jnp.float32)
    d = -2.0 * d
    d = d + jnp.sum(x2 * x2, axis=0)[:, None]
    d = d + jnp.sum(x1 * x1, axis=0)[None, :]
    iota_f = lax.broadcasted_iota(jnp.int32, (S, TK), 0).astype(jnp.float32)
    big = jnp.float32(S)
    recips = []
    idxs = []
    for k in range(3):
        m = jnp.min(d, axis=0)
        ikf = jnp.min(jnp.where(d == m, iota_f, big), axis=0)
        recips.append(1.0 / (m + 1e-8))
        idxs.append(ikf)
        if k < 2:
            d = jnp.where(iota_f == ikf, jnp.float32(jnp.inf), d)
    norm = recips[0] + recips[1] + recips[2]
    idx_ref[...] = jnp.concatenate(
        [ikf.astype(jnp.int32) + b * S for ikf in idxs])
    w_ref[...] = jnp.concatenate([r / norm for r in recips])


def _knn(xyz1, xyz2):
    return pl.pallas_call(
        _knn_body,
        grid=(B, NK),
        in_specs=[
            pl.BlockSpec((1, 3, TK), lambda b, t: (b, 0, t)),
            pl.BlockSpec((1, 3, S), lambda b, t: (b, 0, 0)),
        ],
        out_specs=[
            pl.BlockSpec((TK * 3,), lambda b, t: (b * NK + t,)),
            pl.BlockSpec((TK * 3,), lambda b, t: (b * NK + t,)),
        ],
        out_shape=[
            jax.ShapeDtypeStruct((BN * 3,), jnp.int32),
            jax.ShapeDtypeStruct((BN * 3,), jnp.float32),
        ],
    )(xyz1, xyz2)


_NW = 32
_PW = BN // _NW
_CP = 16
_NCH = _PW // _CP


def _interp_sc(table, idx_flat, w_flat):
    mesh = plsc.VectorSubcoreMesh(core_axis_name="c", subcore_axis_name="s")

    @functools.partial(
        pl.kernel,
        mesh=mesh,
        compiler_params=pltpu.CompilerParams(needs_layout_passes=False),
        out_type=jax.ShapeDtypeStruct((BN, D2), jnp.float32),
        scratch_types=[
            pltpu.VMEM((_PW * 3,), jnp.int32),
            pltpu.VMEM((_PW * 3,), jnp.float32),
            pltpu.VMEM((_CP * 3, D2), jnp.float32),
            pltpu.VMEM((_CP * 3, D2), jnp.float32),
            pltpu.VMEM((_CP, D2), jnp.float32),
            pltpu.VMEM((_CP, D2), jnp.float32),
            pltpu.SemaphoreType.DMA,
            pltpu.SemaphoreType.DMA,
        ],
    )
    def k(table_hbm, idx_hbm, w_hbm, out_hbm, idx_v, w_v,
          rows0, rows1, outb0, outb1, sem0, sem1):
        wid = lax.axis_index("s") * 2 + lax.axis_index("c")
        base = wid * _PW
        pltpu.sync_copy(idx_hbm.at[pl.ds(base * 3, _PW * 3)], idx_v)
        pltpu.sync_copy(w_hbm.at[pl.ds(base * 3, _PW * 3)], w_v)

        def gather(ci, rows, sem):
            for kk in range(3):
                pltpu.async_copy(
                    table_hbm.at[idx_v.at[pl.ds(kk * _PW + ci * _CP, _CP)]],
                    rows.at[pl.ds(kk * _CP, _CP)], sem)

        def compute(ci, rows, outb):
            cb = ci * _CP
            for p in range(_CP):
                w0 = plsc.load_gather(
                    w_v, [jnp.full((16,), 0, jnp.int32) + (cb + p)])
                w1 = plsc.load_gather(
                    w_v, [jnp.full((16,), _PW, jnp.int32) + (cb + p)])
                w2 = plsc.load_gather(
                    w_v, [jnp.full((16,), 2 * _PW, jnp.int32) + (cb + p)])
                for j in range(D2 // 16):
                    sl = pl.ds(j * 16, 16)
                    outb[p, sl] = (rows[p, sl] * w0
                                   + rows[_CP + p, sl] * w1
                                   + rows[2 * _CP + p, sl] * w2)
            pltpu.sync_copy(outb, out_hbm.at[pl.ds(base + cb, _CP)])

        def drain(rows, sem):
            pltpu.make_async_copy(
                table_hbm.at[pl.ds(0, _CP * 3)], rows, sem).wait()

        gather(0, rows0, sem0)
        gather(1, rows1, sem1)

        def pair(i, carry):
            c = 2 * i
            drain(rows0, sem0)
            compute(c, rows0, outb0)
            gather(c + 2, rows0, sem0)
            drain(rows1, sem1)
            compute(c + 1, rows1, outb1)
            gather(c + 3, rows1, sem1)
            return carry

        lax.fori_loop(0, _NCH // 2 - 1, pair, 0)
        drain(rows0, sem0)
        compute(_NCH - 2, rows0, outb0)
        drain(rows1, sem1)
        compute(_NCH - 1, rows1, outb1)

    return k(table, idx_flat, w_flat)


def _m1_body(p1_ref, it_ref, wa_ref, wb_ref, b0_ref, z_ref, s_ref, q_ref):
    b = pl.program_id(0)
    t = pl.program_id(1)
    bf = jnp.bfloat16
    z = (jnp.dot(wa_ref[...].astype(bf), p1_ref[0].astype(bf),
                 preferred_element_type=jnp.float32)
         + lax.dot_general(wb_ref[...].astype(bf), it_ref[...].astype(bf),
                           (((1,), (1,)), ((), ())),
                           preferred_element_type=jnp.float32)
         + b0_ref[...])
    z_ref[0] = z.astype(bf)

    @pl.when((b == 0) & (t == 0))
    def _():
        s_ref[...] = jnp.zeros_like(s_ref)
        q_ref[...] = jnp.zeros_like(q_ref)

    s_ref[...] += jnp.sum(z, axis=1, keepdims=True)
    q_ref[...] += jnp.sum(z * z, axis=1, keepdims=True)


def _m1(points1, interp, wa, wb, b0c):
    return pl.pallas_call(
        _m1_body,
        grid=(B, NT),
        in_specs=[
            pl.BlockSpec((1, D1, TN), lambda b, t: (b, 0, t)),
            pl.BlockSpec((TN, D2), lambda b, t: (b * NT + t, 0)),
            pl.BlockSpec((C1, D1), lambda b, t: (0, 0)),
            pl.BlockSpec((C1, D2), lambda b, t: (0, 0)),
            pl.BlockSpec((C1, 1), lambda b, t: (0, 0)),
        ],
        out_specs=[
            pl.BlockSpec((1, C1, TN), lambda b, t: (b, 0, t)),
            pl.BlockSpec((C1, 1), lambda b, t: (0, 0)),
            pl.BlockSpec((C1, 1), lambda b, t: (0, 0)),
        ],
        out_shape=[
            jax.ShapeDtypeStruct((B, C1, N), jnp.bfloat16),
            jax.ShapeDtypeStruct((C1, 1), jnp.float32),
            jax.ShapeDtypeStruct((C1, 1), jnp.float32),
        ],
    )(points1, interp, wa, wb, b0c)


def _m2_body(z_ref, s_ref, q_ref, g_ref, be_ref, w1_ref, b1_ref,
             z2_ref, s2_ref, q2_ref):
    b = pl.program_id(0)
    t = pl.program_id(1)
    bf = jnp.bfloat16
    mean = s_ref[...] * (1.0 / BN)
    var = q_ref[...] * (1.0 / BN) - mean * mean
    inv = lax.rsqrt(var + 1e-5)
    h = (z_ref[0].astype(jnp.float32) - mean) * (inv * g_ref[...]) + be_ref[...]
    h = jnp.maximum(h, 0.0)
    z2 = jnp.dot(w1_ref[...].astype(bf), h.astype(bf),
                 preferred_element_type=jnp.float32) + b1_ref[...]
    z2_ref[0] = z2.astype(bf)

    @pl.when((b == 0) & (t == 0))
    def _():
        s2_ref[...] = jnp.zeros_like(s2_ref)
        q2_ref[...] = jnp.zeros_like(q2_ref)

    s2_ref[...] += jnp.sum(z2, axis=1, keepdims=True)
    q2_ref[...] += jnp.sum(z2 * z2, axis=1, keepdims=True)


def _m2(z1, s1, q1, g0c, be0c, w1, b1c):
    return pl.pallas_call(
        _m2_body,
        grid=(B, NT),
        in_specs=[
            pl.BlockSpec((1, C1, TN), lambda b, t: (b, 0, t)),
            pl.BlockSpec((C1, 1), lambda b, t: (0, 0)),
            pl.BlockSpec((C1, 1), lambda b, t: (0, 0)),
            pl.BlockSpec((C1, 1), lambda b, t: (0, 0)),
            pl.BlockSpec((C1, 1), lambda b, t: (0, 0)),
            pl.BlockSpec((C2, C1), lambda b, t: (0, 0)),
            pl.BlockSpec((C2, 1), lambda b, t: (0, 0)),
        ],
        out_specs=[
            pl.BlockSpec((1, C2, TN), lambda b, t: (b, 0, t)),
            pl.BlockSpec((C2, 1), lambda b, t: (0, 0)),
            pl.BlockSpec((C2, 1), lambda b, t: (0, 0)),
        ],
        out_shape=[
            jax.ShapeDtypeStruct((B, C2, N), jnp.bfloat16),
            jax.ShapeDtypeStruct((C2, 1), jnp.float32),
            jax.ShapeDtypeStruct((C2, 1), jnp.float32),
        ],
    )(z1, s1, q1, g0c, be0c, w1, b1c)


def _m3_body(z_ref, s_ref, q_ref, g_ref, be_ref, o_ref):
    mean = s_ref[...] * (1.0 / BN)
    var = q_ref[...] * (1.0 / BN) - mean * mean
    inv = lax.rsqrt(var + 1e-5)
    o_ref[0] = jnp.maximum(
        (z_ref[0].astype(jnp.float32) - mean) * (inv * g_ref[...])
        + be_ref[...], 0.0)


def _m3(z2, s2, q2, g1c, be1c):
    return pl.pallas_call(
        _m3_body,
        grid=(B, NT),
        in_specs=[
            pl.BlockSpec((1, C2, TN), lambda b, t: (b, 0, t)),
            pl.BlockSpec((C2, 1), lambda b, t: (0, 0)),
            pl.BlockSpec((C2, 1), lambda b, t: (0, 0)),
            pl.BlockSpec((C2, 1), lambda b, t: (0, 0)),
            pl.BlockSpec((C2, 1), lambda b, t: (0, 0)),
        ],
        out_specs=pl.BlockSpec((1, C2, TN), lambda b, t: (b, 0, t)),
        out_shape=jax.ShapeDtypeStruct((B, C2, N), jnp.float32),
    )(z2, s2, q2, g1c, be1c)


def kernel(xyz1, xyz2, points1, points2, W0, b0, g0, beta0, W1, b1, g1, beta1):
    idx3, w3 = _knn(xyz1, xyz2)
    table = jnp.transpose(points2, (0, 2, 1)).reshape(B * S, D2)
    interp = _interp_sc(table, idx3, w3)
    z1, s1, q1 = _m1(points1, interp, W0[:, :D1], W0[:, D1:],
                     b0.reshape(-1, 1))
    z2, s2, q2 = _m2(z1, s1, q1, g0.reshape(-1, 1), beta0.reshape(-1, 1),
                     W1, b1.reshape(-1, 1))
    return _m3(z2, s2, q2, g1.reshape(-1, 1), beta1.reshape(-1, 1))


# speedup vs baseline: 21.2441x; 1.1655x over previous
import functools

import jax
import jax.numpy as jnp
from jax import lax
from jax.experimental import pallas as pl
from jax.experimental.pallas import tpu as pltpu
from jax.experimental.pallas import tpu_sc as plsc

B, N, S = 8, 4096, 1024
D1, D2 = 128, 256
C1, C2 = 256, 256
BN = B * N

TN = 1024
NT = N // TN
TK = 1024
NK = N // TK


def _knn_body(x1_ref, x2_ref, idx_ref, w_ref):
    b = pl.program_id(0)
    x1 = x1_ref[0]
    x2 = x2_ref[0]
    d = lax.dot_general(x2, x1, (((0,), (0,)), ((), ())),
                        preferred_element_type=jnp.float32)
    d = -2.0 * d
    d = d + jnp.sum(x2 * x2, axis=0)[:, None]
    d = d + jnp.sum(x1 * x1, axis=0)[None, :]
    iota_f = lax.broadcasted_iota(jnp.int32, (S, TK), 0).astype(jnp.float32)
    big = jnp.float32(S)
    recips = []
    idxs = []
    for k in range(3):
        m = jnp.min(d, axis=0)
        ikf = jnp.min(jnp.where(d == m, iota_f, big), axis=0)
        recips.append(1.0 / (m + 1e-8))
        idxs.append(ikf)
        if k < 2:
            d = jnp.where(iota_f == ikf, jnp.float32(jnp.inf), d)
    norm = recips[0] + recips[1] + recips[2]
    idx_ref[...] = jnp.concatenate(
        [ikf.astype(jnp.int32) + b * S for ikf in idxs])
    w_ref[...] = jnp.concatenate([r / norm for r in recips])


def _knn(xyz1, xyz2):
    return pl.pallas_call(
        _knn_body,
        grid=(B, NK),
        in_specs=[
            pl.BlockSpec((1, 3, TK), lambda b, t: (b, 0, t)),
            pl.BlockSpec((1, 3, S), lambda b, t: (b, 0, 0)),
        ],
        out_specs=[
            pl.BlockSpec((TK * 3,), lambda b, t: (b * NK + t,)),
            pl.BlockSpec((TK * 3,), lambda b, t: (b * NK + t,)),
        ],
        out_shape=[
            jax.ShapeDtypeStruct((BN * 3,), jnp.int32),
            jax.ShapeDtypeStruct((BN * 3,), jnp.float32),
        ],
    )(xyz1, xyz2)


_NW = 32
_PW = BN // _NW
_CP = 16
_NCH = _PW // _CP


def _interp_sc(table, idx_flat, w_flat):
    mesh = plsc.VectorSubcoreMesh(core_axis_name="c", subcore_axis_name="s")

    @functools.partial(
        pl.kernel,
        mesh=mesh,
        compiler_params=pltpu.CompilerParams(needs_layout_passes=False),
        out_type=jax.ShapeDtypeStruct((BN, D2), jnp.float32),
        scratch_types=[
            pltpu.VMEM((_PW * 3,), jnp.int32),
            pltpu.VMEM((_PW * 3,), jnp.float32),
            pltpu.VMEM((_CP * 3, D2), jnp.float32),
            pltpu.VMEM((_CP * 3, D2), jnp.float32),
            pltpu.VMEM((_CP, D2), jnp.float32),
            pltpu.VMEM((_CP, D2), jnp.float32),
            pltpu.SemaphoreType.DMA,
            pltpu.SemaphoreType.DMA,
        ],
    )
    def k(table_hbm, idx_hbm, w_hbm, out_hbm, idx_v, w_v,
          rows0, rows1, outb0, outb1, sem0, sem1):
        wid = lax.axis_index("s") * 2 + lax.axis_index("c")
        base = wid * _PW
        pltpu.sync_copy(idx_hbm.at[pl.ds(base * 3, _PW * 3)], idx_v)
        pltpu.sync_copy(w_hbm.at[pl.ds(base * 3, _PW * 3)], w_v)

        def gather(ci, rows, sem):
            for kk in range(3):
                pltpu.async_copy(
                    table_hbm.at[idx_v.at[pl.ds(kk * _PW + ci * _CP, _CP)]],
                    rows.at[pl.ds(kk * _CP, _CP)], sem)

        def compute(ci, rows, outb):
            cb = ci * _CP
            for p in range(_CP):
                w0 = plsc.load_gather(
                    w_v, [jnp.full((16,), 0, jnp.int32) + (cb + p)])
                w1 = plsc.load_gather(
                    w_v, [jnp.full((16,), _PW, jnp.int32) + (cb + p)])
                w2 = plsc.load_gather(
                    w_v, [jnp.full((16,), 2 * _PW, jnp.int32) + (cb + p)])
                for j in range(D2 // 16):
                    sl = pl.ds(j * 16, 16)
                    outb[p, sl] = (rows[p, sl] * w0
                                   + rows[_CP + p, sl] * w1
                                   + rows[2 * _CP + p, sl] * w2)
            pltpu.sync_copy(outb, out_hbm.at[pl.ds(base + cb, _CP)])

        def drain(rows, sem):
            pltpu.make_async_copy(
                table_hbm.at[pl.ds(0, _CP * 3)], rows, sem).wait()

        gather(0, rows0, sem0)
        gather(1, rows1, sem1)

        def pair(i, carry):
            c = 2 * i
            drain(rows0, sem0)
            compute(c, rows0, outb0)
            gather(c + 2, rows0, sem0)
            drain(rows1, sem1)
            compute(c + 1, rows1, outb1)
            gather(c + 3, rows1, sem1)
            return carry

        lax.fori_loop(0, _NCH // 2 - 1, pair, 0)
        drain(rows0, sem0)
        compute(_NCH - 2, rows0, outb0)
        drain(rows1, sem1)
        compute(_NCH - 1, rows1, outb1)

    return k(table, idx_flat, w_flat)


def _m1_body(p1_ref, it_ref, wa_ref, wb_ref, b0_ref, z_ref, s_ref, q_ref):
    b = pl.program_id(0)
    t = pl.program_id(1)
    bf = jnp.bfloat16
    z = (jnp.dot(wa_ref[...].astype(bf), p1_ref[0].astype(bf),
                 preferred_element_type=jnp.float32)
         + lax.dot_general(wb_ref[...].astype(bf), it_ref[...].astype(bf),
                           (((1,), (1,)), ((), ())),
                           preferred_element_type=jnp.float32)
         + b0_ref[...])
    z_ref[0] = z.astype(bf)

    @pl.when((b == 0) & (t == 0))
    def _():
        s_ref[...] = jnp.zeros_like(s_ref)
        q_ref[...] = jnp.zeros_like(q_ref)

    s_ref[...] += jnp.sum(z, axis=1, keepdims=True)
    q_ref[...] += jnp.sum(z * z, axis=1, keepdims=True)


def _m1(points1, interp, wa, wb, b0c):
    return pl.pallas_call(
        _m1_body,
        grid=(B, NT),
        in_specs=[
            pl.BlockSpec((1, D1, TN), lambda b, t: (b, 0, t)),
            pl.BlockSpec((TN, D2), lambda b, t: (b * NT + t, 0)),
            pl.BlockSpec((C1, D1), lambda b, t: (0, 0)),
            pl.BlockSpec((C1, D2), lambda b, t: (0, 0)),
            pl.BlockSpec((C1, 1), lambda b, t: (0, 0)),
        ],
        out_specs=[
            pl.BlockSpec((1, C1, TN), lambda b, t: (b, 0, t)),
            pl.BlockSpec((C1, 1), lambda b, t: (0, 0)),
            pl.BlockSpec((C1, 1), lambda b, t: (0, 0)),
        ],
        out_shape=[
            jax.ShapeDtypeStruct((B, C1, N), jnp.bfloat16),
            jax.ShapeDtypeStruct((C1, 1), jnp.float32),
            jax.ShapeDtypeStruct((C1, 1), jnp.float32),
        ],
    )(points1, interp, wa, wb, b0c)


def _m2_body(z_ref, s_ref, q_ref, g_ref, be_ref, w1_ref, b1_ref,
             z2_ref, s2_ref, q2_ref):
    b = pl.program_id(0)
    t = pl.program_id(1)
    bf = jnp.bfloat16
    mean = s_ref[...] * (1.0 / BN)
    var = q_ref[...] * (1.0 / BN) - mean * mean
    inv = lax.rsqrt(var + 1e-5)
    h = (z_ref[0].astype(jnp.float32) - mean) * (inv * g_ref[...]) + be_ref[...]
    h = jnp.maximum(h, 0.0)
    z2 = jnp.dot(w1_ref[...].astype(bf), h.astype(bf),
                 preferred_element_type=jnp.float32) + b1_ref[...]
    z2_ref[0] = z2.astype(bf)

    @pl.when((b == 0) & (t == 0))
    def _():
        s2_ref[...] = jnp.zeros_like(s2_ref)
        q2_ref[...] = jnp.zeros_like(q2_ref)

    s2_ref[...] += jnp.sum(z2, axis=1, keepdims=True)
    q2_ref[...] += jnp.sum(z2 * z2, axis=1, keepdims=True)


def _m2(z1, s1, q1, g0c, be0c, w1, b1c):
    return pl.pallas_call(
        _m2_body,
        grid=(B, NT),
        in_specs=[
            pl.BlockSpec((1, C1, TN), lambda b, t: (b, 0, t)),
            pl.BlockSpec((C1, 1), lambda b, t: (0, 0)),
            pl.BlockSpec((C1, 1), lambda b, t: (0, 0)),
            pl.BlockSpec((C1, 1), lambda b, t: (0, 0)),
            pl.BlockSpec((C1, 1), lambda b, t: (0, 0)),
            pl.BlockSpec((C2, C1), lambda b, t: (0, 0)),
            pl.BlockSpec((C2, 1), lambda b, t: (0, 0)),
        ],
        out_specs=[
            pl.BlockSpec((1, C2, TN), lambda b, t: (b, 0, t)),
            pl.BlockSpec((C2, 1), lambda b, t: (0, 0)),
            pl.BlockSpec((C2, 1), lambda b, t: (0, 0)),
        ],
        out_shape=[
            jax.ShapeDtypeStruct((B, C2, N), jnp.bfloat16),
            jax.ShapeDtypeStruct((C2, 1), jnp.float32),
            jax.ShapeDtypeStruct((C2, 1), jnp.float32),
        ],
    )(z1, s1, q1, g0c, be0c, w1, b1c)


def _m3_body(z_ref, s_ref, q_ref, g_ref, be_ref, o_ref):
    mean = s_ref[...] * (1.0 / BN)
    var = q_ref[...] * (1.0 / BN) - mean * mean
    inv = lax.rsqrt(var + 1e-5)
    o_ref[0] = jnp.maximum(
        (z_ref[0].astype(jnp.float32) - mean) * (inv * g_ref[...])
        + be_ref[...], 0.0)


def _m3(z2, s2, q2, g1c, be1c):
    return pl.pallas_call(
        _m3_body,
        grid=(B, NT),
        in_specs=[
            pl.BlockSpec((1, C2, TN), lambda b, t: (b, 0, t)),
            pl.BlockSpec((C2, 1), lambda b, t: (0, 0)),
            pl.BlockSpec((C2, 1), lambda b, t: (0, 0)),
            pl.BlockSpec((C2, 1), lambda b, t: (0, 0)),
            pl.BlockSpec((C2, 1), lambda b, t: (0, 0)),
        ],
        out_specs=pl.BlockSpec((1, C2, TN), lambda b, t: (b, 0, t)),
        out_shape=jax.ShapeDtypeStruct((B, C2, N), jnp.float32),
    )(z2, s2, q2, g1c, be1c)


def kernel(xyz1, xyz2, points1, points2, W0, b0, g0, beta0, W1, b1, g1, beta1):
    idx3, w3 = _knn(xyz1, xyz2)
    table = jnp.transpose(points2, (0, 2, 1)).reshape(B * S, D2)
    interp = _interp_sc(table, idx3, w3)
    z1, s1, q1 = _m1(points1, interp, W0[:, :D1], W0[:, D1:],
                     b0.reshape(-1, 1))
    z2, s2, q2 = _m2(z1, s1, q1, g0.reshape(-1, 1), beta0.reshape(-1, 1),
                     W1, b1.reshape(-1, 1))
    return _m3(z2, s2, q2, g1.reshape(-1, 1), beta1.reshape(-1, 1))


# speedup vs baseline: 24.5495x; 1.1556x over previous
import functools

import jax
import jax.numpy as jnp
from jax import lax
from jax.experimental import pallas as pl
from jax.experimental.pallas import tpu as pltpu
from jax.experimental.pallas import tpu_sc as plsc

B, N, S = 8, 4096, 1024
D1, D2 = 128, 256
C1, C2 = 256, 256
BN = B * N

BH = B // 2
BNH = BH * N

TN = 1024
NT = N // TN
TK = 1024
NK = N // TK


def _knn_body(bh, x1_ref, x2_ref, idx_ref, w_ref):
    b = pl.program_id(0)
    x1 = x1_ref[0]
    x2 = x2_ref[0]
    d = lax.dot_general(x2, x1, (((0,), (0,)), ((), ())),
                        preferred_element_type=jnp.float32)
    d = -2.0 * d
    d = d + jnp.sum(x2 * x2, axis=0)[:, None]
    d = d + jnp.sum(x1 * x1, axis=0)[None, :]
    iota_f = lax.broadcasted_iota(jnp.int32, (S, TK), 0).astype(jnp.float32)
    big = jnp.float32(S)
    recips = []
    idxs = []
    for k in range(3):
        m = jnp.min(d, axis=0)
        ikf = jnp.min(jnp.where(d == m, iota_f, big), axis=0)
        recips.append(1.0 / (m + 1e-8))
        idxs.append(ikf)
        if k < 2:
            d = jnp.where(iota_f == ikf, jnp.float32(jnp.inf), d)
    norm = recips[0] + recips[1] + recips[2]
    idx_ref[...] = jnp.concatenate(
        [ikf.astype(jnp.int32) + (b + bh) * S for ikf in idxs])
    w_ref[...] = jnp.concatenate([r / norm for r in recips])


def _knn(xyz1, xyz2, bh):
    return pl.pallas_call(
        functools.partial(_knn_body, bh),
        grid=(BH, NK),
        in_specs=[
            pl.BlockSpec((1, 3, TK), lambda b, t: (b + bh, 0, t)),
            pl.BlockSpec((1, 3, S), lambda b, t: (b + bh, 0, 0)),
        ],
        out_specs=[
            pl.BlockSpec((TK * 3,), lambda b, t: (b * NK + t,)),
            pl.BlockSpec((TK * 3,), lambda b, t: (b * NK + t,)),
        ],
        out_shape=[
            jax.ShapeDtypeStruct((BNH * 3,), jnp.int32),
            jax.ShapeDtypeStruct((BNH * 3,), jnp.float32),
        ],
    )(xyz1, xyz2)


_NW = 32
_PW = BNH // _NW
_CP = 16
_NCH = _PW // _CP


def _interp_sc(table, idx_flat, w_flat):
    mesh = plsc.VectorSubcoreMesh(core_axis_name="c", subcore_axis_name="s")

    @functools.partial(
        pl.kernel,
        mesh=mesh,
        compiler_params=pltpu.CompilerParams(needs_layout_passes=False),
        out_type=jax.ShapeDtypeStruct((BNH, D2), jnp.float32),
        scratch_types=[
            pltpu.VMEM((_PW * 3,), jnp.int32),
            pltpu.VMEM((_PW * 3,), jnp.float32),
            pltpu.VMEM((_CP * 3, D2), jnp.float32),
            pltpu.VMEM((_CP * 3, D2), jnp.float32),
            pltpu.VMEM((_CP, D2), jnp.float32),
            pltpu.VMEM((_CP, D2), jnp.float32),
            pltpu.SemaphoreType.DMA,
            pltpu.SemaphoreType.DMA,
        ],
    )
    def k(table_hbm, idx_hbm, w_hbm, out_hbm, idx_v, w_v,
          rows0, rows1, outb0, outb1, sem0, sem1):
        wid = lax.axis_index("s") * 2 + lax.axis_index("c")
        base = wid * _PW
        blk = wid // 2
        half = wid % 2
        for kk in range(3):
            src = blk * 3 * TK + kk * TK + half * _PW
            pltpu.sync_copy(idx_hbm.at[pl.ds(src, _PW)],
                            idx_v.at[pl.ds(kk * _PW, _PW)])
            pltpu.sync_copy(w_hbm.at[pl.ds(src, _PW)],
                            w_v.at[pl.ds(kk * _PW, _PW)])

        def gather(ci, rows, sem):
            for kk in range(3):
                pltpu.async_copy(
                    table_hbm.at[idx_v.at[pl.ds(kk * _PW + ci * _CP, _CP)]],
                    rows.at[pl.ds(kk * _CP, _CP)], sem)

        def compute(ci, rows, outb):
            cb = ci * _CP
            for p in range(_CP):
                w0 = plsc.load_gather(
                    w_v, [jnp.full((16,), 0, jnp.int32) + (cb + p)])
                w1 = plsc.load_gather(
                    w_v, [jnp.full((16,), _PW, jnp.int32) + (cb + p)])
                w2 = plsc.load_gather(
                    w_v, [jnp.full((16,), 2 * _PW, jnp.int32) + (cb + p)])
                for j in range(D2 // 16):
                    sl = pl.ds(j * 16, 16)
                    outb[p, sl] = (rows[p, sl] * w0
                                   + rows[_CP + p, sl] * w1
                                   + rows[2 * _CP + p, sl] * w2)
            pltpu.sync_copy(outb, out_hbm.at[pl.ds(base + cb, _CP)])

        def drain(rows, sem):
            pltpu.make_async_copy(
                table_hbm.at[pl.ds(0, _CP * 3)], rows, sem).wait()

        gather(0, rows0, sem0)
        gather(1, rows1, sem1)

        def pair(i, carry):
            c = 2 * i
            drain(rows0, sem0)
            compute(c, rows0, outb0)
            gather(c + 2, rows0, sem0)
            drain(rows1, sem1)
            compute(c + 1, rows1, outb1)
            gather(c + 3, rows1, sem1)
            return carry

        lax.fori_loop(0, _NCH // 2 - 1, pair, 0)
        drain(rows0, sem0)
        compute(_NCH - 2, rows0, outb0)
        drain(rows1, sem1)
        compute(_NCH - 1, rows1, outb1)

    return k(table, idx_flat, w_flat)


def _m1_body(bh, p1_ref, it_ref, wa_ref, wb_ref, b0_ref, z_ref, s_ref, q_ref):
    b = pl.program_id(0)
    t = pl.program_id(1)
    bf = jnp.bfloat16
    z = (jnp.dot(wa_ref[...].astype(bf), p1_ref[0].astype(bf),
                 preferred_element_type=jnp.float32)
         + lax.dot_general(wb_ref[...].astype(bf), it_ref[...].astype(bf),
                           (((1,), (1,)), ((), ())),
                           preferred_element_type=jnp.float32)
         + b0_ref[...])
    z_ref[0] = z.astype(bf)

    @pl.when((b == 0) & (t == 0))
    def _():
        s_ref[...] = jnp.zeros_like(s_ref)
        q_ref[...] = jnp.zeros_like(q_ref)

    s_ref[...] += jnp.sum(z, axis=1, keepdims=True)
    q_ref[...] += jnp.sum(z * z, axis=1, keepdims=True)


def _m1(points1, interp, wa, wb, b0c, bh):
    return pl.pallas_call(
        functools.partial(_m1_body, bh),
        grid=(BH, NT),
        in_specs=[
            pl.BlockSpec((1, D1, TN), lambda b, t: (b + bh, 0, t)),
            pl.BlockSpec((TN, D2), lambda b, t: (b * NT + t, 0)),
            pl.BlockSpec((C1, D1), lambda b, t: (0, 0)),
            pl.BlockSpec((C1, D2), lambda b, t: (0, 0)),
            pl.BlockSpec((C1, 1), lambda b, t: (0, 0)),
        ],
        out_specs=[
            pl.BlockSpec((1, C1, TN), lambda b, t: (b, 0, t)),
            pl.BlockSpec((C1, 1), lambda b, t: (0, 0)),
            pl.BlockSpec((C1, 1), lambda b, t: (0, 0)),
        ],
        out_shape=[
            jax.ShapeDtypeStruct((BH, C1, N), jnp.bfloat16),
            jax.ShapeDtypeStruct((C1, 1), jnp.float32),
            jax.ShapeDtypeStruct((C1, 1), jnp.float32),
        ],
    )(points1, interp, wa, wb, b0c)


def _m2_body(z_ref, sa_ref, qa_ref, sb_ref, qb_ref, g_ref, be_ref, w1_ref,
             b1_ref, z2_ref, s2_ref, q2_ref):
    b = pl.program_id(0)
    t = pl.program_id(1)
    bf = jnp.bfloat16
    mean = (sa_ref[...] + sb_ref[...]) * (1.0 / BN)
    var = (qa_ref[...] + qb_ref[...]) * (1.0 / BN) - mean * mean
    inv = lax.rsqrt(var + 1e-5)
    h = (z_ref[0].astype(jnp.float32) - mean) * (inv * g_ref[...]) + be_ref[...]
    h = jnp.maximum(h, 0.0)
    z2 = jnp.dot(w1_ref[...].astype(bf), h.astype(bf),
                 preferred_element_type=jnp.float32) + b1_ref[...]
    z2_ref[0] = z2.astype(bf)

    @pl.when((b == 0) & (t == 0))
    def _():
        s2_ref[...] = jnp.zeros_like(s2_ref)
        q2_ref[...] = jnp.zeros_like(q2_ref)

    s2_ref[...] += jnp.sum(z2, axis=1, keepdims=True)
    q2_ref[...] += jnp.sum(z2 * z2, axis=1, keepdims=True)


def _m2(z1, s1a, q1a, s1b, q1b, g0c, be0c, w1, b1c):
    cvec = pl.BlockSpec((C1, 1), lambda b, t: (0, 0))
    return pl.pallas_call(
        _m2_body,
        grid=(BH, NT),
        in_specs=[
            pl.BlockSpec((1, C1, TN), lambda b, t: (b, 0, t)),
            cvec, cvec, cvec, cvec, cvec, cvec,
            pl.BlockSpec((C2, C1), lambda b, t: (0, 0)),
            pl.BlockSpec((C2, 1), lambda b, t: (0, 0)),
        ],
        out_specs=[
            pl.BlockSpec((1, C2, TN), lambda b, t: (b, 0, t)),
            pl.BlockSpec((C2, 1), lambda b, t: (0, 0)),
            pl.BlockSpec((C2, 1), lambda b, t: (0, 0)),
        ],
        out_shape=[
            jax.ShapeDtypeStruct((BH, C2, N), jnp.bfloat16),
            jax.ShapeDtypeStruct((C2, 1), jnp.float32),
            jax.ShapeDtypeStruct((C2, 1), jnp.float32),
        ],
    )(z1, s1a, q1a, s1b, q1b, g0c, be0c, w1, b1c)


def _m3_body(z_ref, sa_ref, qa_ref, sb_ref, qb_ref, g_ref, be_ref, o_ref):
    mean = (sa_ref[...] + sb_ref[...]) * (1.0 / BN)
    var = (qa_ref[...] + qb_ref[...]) * (1.0 / BN) - mean * mean
    inv = lax.rsqrt(var + 1e-5)
    o_ref[0] = jnp.maximum(
        (z_ref[0].astype(jnp.float32) - mean) * (inv * g_ref[...])
        + be_ref[...], 0.0)


def _m3a(z2, s2a, q2a, s2b, q2b, g1c, be1c):
    cvec = pl.BlockSpec((C2, 1), lambda b, t: (0, 0))
    return pl.pallas_call(
        _m3_body,
        grid=(BH, NT),
        in_specs=[
            pl.BlockSpec((1, C2, TN), lambda b, t: (b, 0, t)),
            cvec, cvec, cvec, cvec, cvec, cvec,
        ],
        out_specs=pl.BlockSpec((1, C2, TN), lambda b, t: (b, 0, t)),
        out_shape=jax.ShapeDtypeStruct((B, C2, N), jnp.float32),
    )(z2, s2a, q2a, s2b, q2b, g1c, be1c)


def _m3b_body(o_in_ref, z_ref, sa_ref, qa_ref, sb_ref, qb_ref, g_ref, be_ref,
              o_ref):
    _m3_body(z_ref, sa_ref, qa_ref, sb_ref, qb_ref, g_ref, be_ref, o_ref)


def _m3b(out0, z2, s2a, q2a, s2b, q2b, g1c, be1c):
    cvec = pl.BlockSpec((C2, 1), lambda b, t: (0, 0))
    return pl.pallas_call(
        _m3b_body,
        grid=(BH, NT),
        in_specs=[
            pl.BlockSpec(memory_space=pl.ANY),
            pl.BlockSpec((1, C2, TN), lambda b, t: (b, 0, t)),
            cvec, cvec, cvec, cvec, cvec, cvec,
        ],
        out_specs=pl.BlockSpec((1, C2, TN), lambda b, t: (b + BH, 0, t)),
        out_shape=jax.ShapeDtypeStruct((B, C2, N), jnp.float32),
        input_output_aliases={0: 0},
    )(out0, z2, s2a, q2a, s2b, q2b, g1c, be1c)


def kernel(xyz1, xyz2, points1, points2, W0, b0, g0, beta0, W1, b1, g1, beta1):
    table = jnp.transpose(points2, (0, 2, 1)).reshape(B * S, D2)
    idxa, wta = _knn(xyz1, xyz2, 0)
    ga = _interp_sc(table, idxa, wta)
    idxb, wtb = _knn(xyz1, xyz2, BH)
    gb = _interp_sc(table, idxb, wtb)
    wa, wb = W0[:, :D1], W0[:, D1:]
    b0c = b0.reshape(-1, 1)
    z1a, s1a, q1a = _m1(points1, ga, wa, wb, b0c, 0)
    z1b, s1b, q1b = _m1(points1, gb, wa, wb, b0c, BH)
    g0c, be0c = g0.reshape(-1, 1), beta0.reshape(-1, 1)
    b1c = b1.reshape(-1, 1)
    z2a, s2a, q2a = _m2(z1a, s1a, q1a, s1b, q1b, g0c, be0c, W1, b1c)
    z2b, s2b, q2b = _m2(z1b, s1a, q1a, s1b, q1b, g0c, be0c, W1, b1c)
    g1c, be1c = g1.reshape(-1, 1), beta1.reshape(-1, 1)
    out0 = _m3a(z2a, s2a, q2a, s2b, q2b, g1c, be1c)
    return _m3b(out0, z2b, s2a, q2a, s2b, q2b, g1c, be1c)


# speedup vs baseline: 24.7655x; 1.0088x over previous
import functools

import jax
import jax.numpy as jnp
from jax import lax
from jax.experimental import pallas as pl
from jax.experimental.pallas import tpu as pltpu
from jax.experimental.pallas import tpu_sc as plsc

B, N, S = 8, 4096, 1024
D1, D2 = 128, 256
C1, C2 = 256, 256
BN = B * N

BQ = 2
BNQ = BQ * N

TN = 1024
NT = N // TN
TK = 1024
NK = N // TK


def _knn_body(bh, x1_ref, x2_ref, idx_ref, w_ref):
    b = pl.program_id(0)
    x1 = x1_ref[0]
    x2 = x2_ref[0]
    d = lax.dot_general(x2, x1, (((0,), (0,)), ((), ())),
                        preferred_element_type=jnp.float32)
    d = -2.0 * d
    d = d + jnp.sum(x2 * x2, axis=0)[:, None]
    d = d + jnp.sum(x1 * x1, axis=0)[None, :]
    iota_f = lax.broadcasted_iota(jnp.int32, (S, TK), 0).astype(jnp.float32)
    big = jnp.float32(S)
    recips = []
    idxs = []
    for k in range(3):
        m = jnp.min(d, axis=0)
        ikf = jnp.min(jnp.where(d == m, iota_f, big), axis=0)
        recips.append(1.0 / (m + 1e-8))
        idxs.append(ikf)
        if k < 2:
            d = jnp.where(iota_f == ikf, jnp.float32(jnp.inf), d)
    norm = recips[0] + recips[1] + recips[2]
    idx_ref[...] = jnp.concatenate(
        [ikf.astype(jnp.int32) + (b + bh) * S for ikf in idxs])
    w_ref[...] = jnp.concatenate([r / norm for r in recips])


def _knn(xyz1, xyz2, bh):
    return pl.pallas_call(
        functools.partial(_knn_body, bh),
        grid=(BQ, NK),
        in_specs=[
            pl.BlockSpec((1, 3, TK), lambda b, t: (b + bh, 0, t)),
            pl.BlockSpec((1, 3, S), lambda b, t: (b + bh, 0, 0)),
        ],
        out_specs=[
            pl.BlockSpec((TK * 3,), lambda b, t: (b * NK + t,)),
            pl.BlockSpec((TK * 3,), lambda b, t: (b * NK + t,)),
        ],
        out_shape=[
            jax.ShapeDtypeStruct((BNQ * 3,), jnp.int32),
            jax.ShapeDtypeStruct((BNQ * 3,), jnp.float32),
        ],
    )(xyz1, xyz2)


_NW = 32
_PW = BNQ // _NW
_CP = 16
_NCH = _PW // _CP


def _interp_sc(table, idx_flat, w_flat):
    mesh = plsc.VectorSubcoreMesh(core_axis_name="c", subcore_axis_name="s")

    @functools.partial(
        pl.kernel,
        mesh=mesh,
        compiler_params=pltpu.CompilerParams(needs_layout_passes=False),
        out_type=jax.ShapeDtypeStruct((BNQ, D2), jnp.float32),
        scratch_types=[
            pltpu.VMEM((_PW * 3,), jnp.int32),
            pltpu.VMEM((_PW * 3,), jnp.float32),
            pltpu.VMEM((_CP * 3, D2), jnp.float32),
            pltpu.VMEM((_CP * 3, D2), jnp.float32),
            pltpu.VMEM((_CP, D2), jnp.float32),
            pltpu.VMEM((_CP, D2), jnp.float32),
            pltpu.SemaphoreType.DMA,
            pltpu.SemaphoreType.DMA,
        ],
    )
    def k(table_hbm, idx_hbm, w_hbm, out_hbm, idx_v, w_v,
          rows0, rows1, outb0, outb1, sem0, sem1):
        wid = lax.axis_index("s") * 2 + lax.axis_index("c")
        base = wid * _PW
        ppb = TK // _PW
        blk = wid // ppb
        part = wid % ppb
        for kk in range(3):
            src = blk * 3 * TK + kk * TK + part * _PW
            pltpu.sync_copy(idx_hbm.at[pl.ds(src, _PW)],
                            idx_v.at[pl.ds(kk * _PW, _PW)])
            pltpu.sync_copy(w_hbm.at[pl.ds(src, _PW)],
                            w_v.at[pl.ds(kk * _PW, _PW)])

        def gather(ci, rows, sem):
            for kk in range(3):
                pltpu.async_copy(
                    table_hbm.at[idx_v.at[pl.ds(kk * _PW + ci * _CP, _CP)]],
                    rows.at[pl.ds(kk * _CP, _CP)], sem)

        def compute(ci, rows, outb):
            cb = ci * _CP
            for p in range(_CP):
                w0 = plsc.load_gather(
                    w_v, [jnp.full((16,), 0, jnp.int32) + (cb + p)])
                w1 = plsc.load_gather(
                    w_v, [jnp.full((16,), _PW, jnp.int32) + (cb + p)])
                w2 = plsc.load_gather(
                    w_v, [jnp.full((16,), 2 * _PW, jnp.int32) + (cb + p)])
                for j in range(D2 // 16):
                    sl = pl.ds(j * 16, 16)
                    outb[p, sl] = (rows[p, sl] * w0
                                   + rows[_CP + p, sl] * w1
                                   + rows[2 * _CP + p, sl] * w2)
            pltpu.sync_copy(outb, out_hbm.at[pl.ds(base + cb, _CP)])

        def drain(rows, sem):
            pltpu.make_async_copy(
                table_hbm.at[pl.ds(0, _CP * 3)], rows, sem).wait()

        gather(0, rows0, sem0)
        gather(1, rows1, sem1)

        def pair(i, carry):
            c = 2 * i
            drain(rows0, sem0)
            compute(c, rows0, outb0)
            gather(c + 2, rows0, sem0)
            drain(rows1, sem1)
            compute(c + 1, rows1, outb1)
            gather(c + 3, rows1, sem1)
            return carry

        lax.fori_loop(0, _NCH // 2 - 1, pair, 0)
        drain(rows0, sem0)
        compute(_NCH - 2, rows0, outb0)
        drain(rows1, sem1)
        compute(_NCH - 1, rows1, outb1)

    return k(table, idx_flat, w_flat)


def _m1_body(bh, p1_ref, it_ref, wa_ref, wb_ref, b0_ref, z_ref, s_ref, q_ref):
    b = pl.program_id(0)
    t = pl.program_id(1)
    bf = jnp.bfloat16
    z = (jnp.dot(wa_ref[...].astype(bf), p1_ref[0].astype(bf),
                 preferred_element_type=jnp.float32)
         + lax.dot_general(wb_ref[...].astype(bf), it_ref[...].astype(bf),
                           (((1,), (1,)), ((), ())),
                           preferred_element_type=jnp.float32)
         + b0_ref[...])
    z_ref[0] = z.astype(bf)

    @pl.when((b == 0) & (t == 0))
    def _():
        s_ref[...] = jnp.zeros_like(s_ref)
        q_ref[...] = jnp.zeros_like(q_ref)

    s_ref[...] += jnp.sum(z, axis=1, keepdims=True)
    q_ref[...] += jnp.sum(z * z, axis=1, keepdims=True)


def _m1_first_body(bh, p1_ref, it_ref, wa_ref, wb_ref, b0_ref,
                   z_ref, s_ref, q_ref):
    _m1_body(bh, p1_ref, it_ref, wa_ref, wb_ref, b0_ref, z_ref, s_ref, q_ref)


def _m1_next_body(bh, zin_ref, p1_ref, it_ref, wa_ref, wb_ref, b0_ref,
                  z_ref, s_ref, q_ref):
    _m1_body(bh, p1_ref, it_ref, wa_ref, wb_ref, b0_ref, z_ref, s_ref, q_ref)


def _m1(points1, interp, wa, wb, b0c, bh, zin):
    out_specs = [
        pl.BlockSpec((1, C1, TN), lambda b, t: (b + bh, 0, t)),
        pl.BlockSpec((C1, 1), lambda b, t: (0, 0)),
        pl.BlockSpec((C1, 1), lambda b, t: (0, 0)),
    ]
    out_shape = [
        jax.ShapeDtypeStruct((B, C1, N), jnp.bfloat16),
        jax.ShapeDtypeStruct((C1, 1), jnp.float32),
        jax.ShapeDtypeStruct((C1, 1), jnp.float32),
    ]
    in_specs = [
        pl.BlockSpec((1, D1, TN), lambda b, t: (b + bh, 0, t)),
        pl.BlockSpec((TN, D2), lambda b, t: (b * NT + t, 0)),
        pl.BlockSpec((C1, D1), lambda b, t: (0, 0)),
        pl.BlockSpec((C1, D2), lambda b, t: (0, 0)),
        pl.BlockSpec((C1, 1), lambda b, t: (0, 0)),
    ]
    if zin is None:
        return pl.pallas_call(
            functools.partial(_m1_first_body, bh),
            grid=(BQ, NT), in_specs=in_specs,
            out_specs=out_specs, out_shape=out_shape,
        )(points1, interp, wa, wb, b0c)
    return pl.pallas_call(
        functools.partial(_m1_next_body, bh),
        grid=(BQ, NT),
        in_specs=[pl.BlockSpec(memory_space=pl.ANY)] + in_specs,
        out_specs=out_specs, out_shape=out_shape,
        input_output_aliases={0: 0},
    )(zin, points1, interp, wa, wb, b0c)


def _m2_body(z_ref, s_ref, q_ref, g_ref, be_ref, w1_ref, b1_ref,
             z2_ref, s2_ref, q2_ref):
    b = pl.program_id(0)
    t = pl.program_id(1)
    bf = jnp.bfloat16
    mean = s_ref[...] * (1.0 / BN)
    var = q_ref[...] * (1.0 / BN) - mean * mean
    inv = lax.rsqrt(var + 1e-5)
    h = (z_ref[0].astype(jnp.float32) - mean) * (inv * g_ref[...]) + be_ref[...]
    h = jnp.maximum(h, 0.0)
    z2 = jnp.dot(w1_ref[...].astype(bf), h.astype(bf),
                 preferred_element_type=jnp.float32) + b1_ref[...]
    z2_ref[0] = z2.astype(bf)

    @pl.when((b == 0) & (t == 0))
    def _():
        s2_ref[...] = jnp.zeros_like(s2_ref)
        q2_ref[...] = jnp.zeros_like(q2_ref)

    s2_ref[...] += jnp.sum(z2, axis=1, keepdims=True)
    q2_ref[...] += jnp.sum(z2 * z2, axis=1, keepdims=True)


def _m2(z1, s1, q1, g0c, be0c, w1, b1c):
    cvec = pl.BlockSpec((C1, 1), lambda b, t: (0, 0))
    return pl.pallas_call(
        _m2_body,
        grid=(B, NT),
        in_specs=[
            pl.BlockSpec((1, C1, TN), lambda b, t: (b, 0, t)),
            cvec, cvec, cvec, cvec,
            pl.BlockSpec((C2, C1), lambda b, t: (0, 0)),
            pl.BlockSpec((C2, 1), lambda b, t: (0, 0)),
        ],
        out_specs=[
            pl.BlockSpec((1, C2, TN), lambda b, t: (b, 0, t)),
            pl.BlockSpec((C2, 1), lambda b, t: (0, 0)),
            pl.BlockSpec((C2, 1), lambda b, t: (0, 0)),
        ],
        out_shape=[
            jax.ShapeDtypeStruct((B, C2, N), jnp.bfloat16),
            jax.ShapeDtypeStruct((C2, 1), jnp.float32),
            jax.ShapeDtypeStruct((C2, 1), jnp.float32),
        ],
    )(z1, s1, q1, g0c, be0c, w1, b1c)


def _m3_body(z_ref, s_ref, q_ref, g_ref, be_ref, o_ref):
    mean = s_ref[...] * (1.0 / BN)
    var = q_ref[...] * (1.0 / BN) - mean * mean
    inv = lax.rsqrt(var + 1e-5)
    o_ref[0] = jnp.maximum(
        (z_ref[0].astype(jnp.float32) - mean) * (inv * g_ref[...])
        + be_ref[...], 0.0)


def _m3(z2, s2, q2, g1c, be1c):
    cvec = pl.BlockSpec((C2, 1), lambda b, t: (0, 0))
    return pl.pallas_call(
        _m3_body,
        grid=(B, NT),
        in_specs=[
            pl.BlockSpec((1, C2, TN), lambda b, t: (b, 0, t)),
            cvec, cvec, cvec, cvec,
        ],
        out_specs=pl.BlockSpec((1, C2, TN), lambda b, t: (b, 0, t)),
        out_shape=jax.ShapeDtypeStruct((B, C2, N), jnp.float32),
    )(z2, s2, q2, g1c, be1c)


def kernel(xyz1, xyz2, points1, points2, W0, b0, g0, beta0, W1, b1, g1, beta1):
    table = jnp.transpose(points2, (0, 2, 1)).reshape(B * S, D2)
    gs = []
    for qi in range(B // BQ):
        idx, wt = _knn(xyz1, xyz2, BQ * qi)
        gs.append(_interp_sc(table, idx, wt))
    wa, wb = W0[:, :D1], W0[:, D1:]
    b0c = b0.reshape(-1, 1)
    z1 = None
    ss, qs = [], []
    for qi in range(B // BQ):
        z1, s, q = _m1(points1, gs[qi], wa, wb, b0c, BQ * qi, z1)
        ss.append(s)
        qs.append(q)
    s1 = ss[0] + ss[1] + ss[2] + ss[3]
    q1 = qs[0] + qs[1] + qs[2] + qs[3]
    g0c, be0c = g0.reshape(-1, 1), beta0.reshape(-1, 1)
    b1c = b1.reshape(-1, 1)
    z2, s2, q2 = _m2(z1, s1, q1, g0c, be0c, W1, b1c)
    g1c, be1c = g1.reshape(-1, 1), beta1.reshape(-1, 1)
    return _m3(z2, s2, q2, g1c, be1c)
